# baseline (speedup 1.0000x reference)
import functools

import jax
import jax.numpy as jnp
from jax import lax
from jax.experimental import pallas as pl
from jax.experimental.pallas import tpu as pltpu
from jax.experimental.pallas import tpu_sc as plsc

NUM_EXPERTS = 8
TOP_K = 2
N_TOKENS = 2048
D_IN = 768
D_OUT = 768

BLKG = 256
XS_ROWS = 4096 + NUM_EXPERTS * BLKG - 256
XS_ROWS = ((4096 + NUM_EXPERTS * (BLKG - 1)) + BLKG - 1) // BLKG * BLKG
G_BLOCKS = XS_ROWS // BLKG

NW = 32
TOK_W = N_TOKENS // NW
ROW_SLICES = D_OUT // 16


def _plan_body(x_ref, ltri_ref, gw_ref, gb_ref, gamma_ref, beta_ref,
               p0_ref, p1_ref, w0_ref, w1_ref, be_ref, nb_ref):
    x = x_ref[...]
    logits = jnp.dot(x, gw_ref[...],
                     preferred_element_type=jnp.float32) + gb_ref[...]
    gx = jnp.sqrt(jnp.sum(logits * logits, axis=1, keepdims=True))
    nx = gx / (jnp.mean(gx, axis=0, keepdims=True) + 1e-06)
    logits = gamma_ref[...] * (logits * nx) + beta_ref[...] + logits
    m = jnp.max(logits, axis=1, keepdims=True)
    ex = jnp.exp(logits - m)
    gates = ex / jnp.sum(ex, axis=1, keepdims=True)
    ids = jax.lax.broadcasted_iota(jnp.int32, gates.shape, 1)
    m1 = jnp.max(gates, axis=1, keepdims=True)
    i1 = jnp.min(jnp.where(gates == m1, ids, NUM_EXPERTS),
                 axis=1, keepdims=True)
    sel1 = ids == i1
    g2 = jnp.where(sel1, -jnp.inf, gates)
    m2 = jnp.max(g2, axis=1, keepdims=True)
    i2 = jnp.min(jnp.where(g2 == m2, ids, NUM_EXPERTS),
                 axis=1, keepdims=True)
    sel2 = ids == i2
    w0_ref[...] = jnp.broadcast_to(m1, (N_TOKENS, 16))
    w1_ref[...] = jnp.broadcast_to(m2, (N_TOKENS, 16))

    s = jnp.where(sel1, 1.0, 0.0) + jnp.where(sel2, 1.0, 0.0)
    p = jnp.dot(ltri_ref[...], s.astype(jnp.bfloat16),
                preferred_element_type=jnp.float32)
    cnt = jnp.sum(s, axis=0, keepdims=True)
    pcnt = jnp.ceil(cnt / BLKG) * BLKG
    er = jax.lax.broadcasted_iota(jnp.int32, (NUM_EXPERTS, NUM_EXPERTS), 0)
    ec = jax.lax.broadcasted_iota(jnp.int32, (NUM_EXPERTS, NUM_EXPERTS), 1)
    sut = jnp.where(er < ec, 1.0, 0.0)
    offs = jnp.dot(pcnt, sut, preferred_element_type=jnp.float32)
    a = offs + p
    p0_ref[...] = jnp.sum(jnp.where(sel1, a, 0.0), axis=1,
                          keepdims=True).astype(jnp.int32)
    p1_ref[...] = jnp.sum(jnp.where(sel2, a, 0.0), axis=1,
                          keepdims=True).astype(jnp.int32)

    gid = jax.lax.broadcasted_iota(jnp.int32, (1, G_BLOCKS), 1)
    eids = jax.lax.broadcasted_iota(jnp.int32, (1, NUM_EXPERTS), 1)
    blk = jnp.zeros((1, G_BLOCKS), jnp.float32)
    for e in range(NUM_EXPERTS):
        sel_e = jnp.where(eids == e, 1.0, 0.0)
        offs_e = jnp.sum(offs * sel_e)
        pcnt_e = jnp.sum(pcnt * sel_e)
        lo = (offs_e / BLKG).astype(jnp.int32)
        hi = ((offs_e + pcnt_e) / BLKG).astype(jnp.int32)
        blk = blk + jnp.where((gid >= lo) & (gid < hi), float(e), 0.0)
    be_ref[...] = blk.astype(jnp.int32)
    nb_ref[...] = (jnp.sum(pcnt, keepdims=True) / BLKG).astype(jnp.int32)


def _plan(x, ltri, gate_W, gate_b, gamma, beta):
    return pl.pallas_call(
        _plan_body,
        grid=(1,),
        in_specs=[
            pl.BlockSpec((N_TOKENS, D_IN), lambda i: (0, 0)),
            pl.BlockSpec((N_TOKENS, N_TOKENS), lambda i: (0, 0)),
            pl.BlockSpec((D_IN, NUM_EXPERTS), lambda i: (0, 0)),
            pl.BlockSpec((NUM_EXPERTS,), lambda i: (0,)),
            pl.BlockSpec((1, NUM_EXPERTS), lambda i: (0, 0)),
            pl.BlockSpec((1, NUM_EXPERTS), lambda i: (0, 0)),
        ],
        out_specs=[
            pl.BlockSpec((N_TOKENS, 1), lambda i: (0, 0)),
            pl.BlockSpec((N_TOKENS, 1), lambda i: (0, 0)),
            pl.BlockSpec((N_TOKENS, 16), lambda i: (0, 0)),
            pl.BlockSpec((N_TOKENS, 16), lambda i: (0, 0)),
            pl.BlockSpec((1, G_BLOCKS), lambda i: (0, 0)),
            pl.BlockSpec((1, 1), lambda i: (0, 0)),
        ],
        out_shape=[
            jax.ShapeDtypeStruct((N_TOKENS, 1), jnp.int32),
            jax.ShapeDtypeStruct((N_TOKENS, 1), jnp.int32),
            jax.ShapeDtypeStruct((N_TOKENS, 16), jnp.float32),
            jax.ShapeDtypeStruct((N_TOKENS, 16), jnp.float32),
            jax.ShapeDtypeStruct((1, G_BLOCKS), jnp.int32),
            jax.ShapeDtypeStruct((1, 1), jnp.int32),
        ],
    )(x, ltri, gate_W, gate_b, gamma, beta)


@functools.lru_cache(maxsize=None)
def _sc_dispatch_kernel():
    mesh = plsc.VectorSubcoreMesh(core_axis_name="c", subcore_axis_name="s")

    @functools.partial(
        pl.kernel,
        out_type=jax.ShapeDtypeStruct((XS_ROWS, D_IN), jnp.float32),
        mesh=mesh,
        scratch_types=[
            pltpu.VMEM((TOK_W, D_IN), jnp.float32),
            pltpu.VMEM((TOK_W,), jnp.int32),
            pltpu.VMEM((TOK_W,), jnp.int32),
            pltpu.SemaphoreType.DMA,
        ],
    )
    def _sc_dispatch(x_hbm, p0_hbm, p1_hbm, xs_hbm, xrows, idx0, idx1, sem):
        wid = lax.axis_index("s") * 2 + lax.axis_index("c")
        base = wid * TOK_W
        pltpu.sync_copy(x_hbm.at[pl.ds(base, TOK_W)], xrows)
        pltpu.sync_copy(p0_hbm.at[pl.ds(base, TOK_W)], idx0)
        pltpu.sync_copy(p1_hbm.at[pl.ds(base, TOK_W)], idx1)
        pltpu.async_copy(xrows, xs_hbm.at[idx0], sem).wait()
        pltpu.async_copy(xrows, xs_hbm.at[idx1], sem).wait()

    return _sc_dispatch


def _gmm_body(be_sref, nb_sref, xs_ref, w_ref, b_ref, y_ref):
    g = pl.program_id(0)

    @pl.when(g < nb_sref[0])
    def _():
        xbf = xs_ref[...].astype(jnp.bfloat16)
        wbf = w_ref[0].astype(jnp.bfloat16)
        y_ref[...] = jnp.dot(
            xbf, wbf, preferred_element_type=jnp.float32) + b_ref[0]


def _gmm(blk2exp, nblk, xs, W, b3):
    spec = pltpu.PrefetchScalarGridSpec(
        num_scalar_prefetch=2,
        grid=(G_BLOCKS,),
        in_specs=[
            pl.BlockSpec((BLKG, D_IN), lambda g, be, nb: (g, 0)),
            pl.BlockSpec((1, D_IN, D_OUT), lambda g, be, nb: (be[g], 0, 0)),
            pl.BlockSpec((1, 1, D_OUT), lambda g, be, nb: (be[g], 0, 0)),
        ],
        out_specs=pl.BlockSpec((BLKG, D_OUT), lambda g, be, nb: (g, 0)),
    )
    return pl.pallas_call(
        _gmm_body,
        grid_spec=spec,
        out_shape=jax.ShapeDtypeStruct((XS_ROWS, D_OUT), jnp.float32),
    )(blk2exp, nblk, xs, W, b3)


@functools.lru_cache(maxsize=None)
def _sc_combine_kernel():
    mesh = plsc.VectorSubcoreMesh(core_axis_name="c", subcore_axis_name="s")

    @functools.partial(
        pl.kernel,
        out_type=jax.ShapeDtypeStruct((N_TOKENS, D_OUT), jnp.float32),
        mesh=mesh,
        scratch_types=[
            pltpu.VMEM((TOK_W, D_OUT), jnp.float32),
            pltpu.VMEM((TOK_W, D_OUT), jnp.float32),
            pltpu.VMEM((TOK_W,), jnp.int32),
            pltpu.VMEM((TOK_W,), jnp.int32),
            pltpu.VMEM((TOK_W, 16), jnp.float32),
            pltpu.VMEM((TOK_W, 16), jnp.float32),
            pltpu.SemaphoreType.DMA,
        ],
    )
    def _sc_combine(y_hbm, p0_hbm, p1_hbm, w0_hbm, w1_hbm, out_hbm,
                    buf0, buf1, idx0, idx1, w0v, w1v, sem):
        wid = lax.axis_index("s") * 2 + lax.axis_index("c")
        base = wid * TOK_W
        pltpu.sync_copy(p0_hbm.at[pl.ds(base, TOK_W)], idx0)
        pltpu.sync_copy(p1_hbm.at[pl.ds(base, TOK_W)], idx1)
        pltpu.sync_copy(w0_hbm.at[pl.ds(base, TOK_W)], w0v)
        pltpu.sync_copy(w1_hbm.at[pl.ds(base, TOK_W)], w1v)
        pltpu.async_copy(y_hbm.at[idx0], buf0, sem).wait()
        pltpu.async_copy(y_hbm.at[idx1], buf1, sem).wait()

        def row(i, carry):
            w0i = w0v[i, pl.ds(0, 16)]
            w1i = w1v[i, pl.ds(0, 16)]
            for j in range(ROW_SLICES):
                sl = pl.ds(j * 16, 16)
                buf0[i, sl] = w0i * buf0[i, sl] + w1i * buf1[i, sl]
            return carry

        lax.fori_loop(0, TOK_W, row, 0)
        pltpu.sync_copy(buf0, out_hbm.at[pl.ds(base, TOK_W)])

    return _sc_combine


@jax.jit
def kernel(x, W, b, gate_W, gate_b, gamma, beta):
    rows = jax.lax.broadcasted_iota(jnp.int32, (N_TOKENS, N_TOKENS), 0)
    cols = jax.lax.broadcasted_iota(jnp.int32, (N_TOKENS, N_TOKENS), 1)
    ltri = jnp.where(cols < rows, 1.0, 0.0).astype(jnp.bfloat16)
    p0, p1, w0, w1, be, nb = _plan(x, ltri, gate_W, gate_b, gamma, beta)
    p0f = p0.reshape(N_TOKENS)
    p1f = p1.reshape(N_TOKENS)
    xs = _sc_dispatch_kernel()(x, p0f, p1f)
    ybuf = _gmm(be.reshape(G_BLOCKS), nb.reshape(1), xs,
                W, b.reshape(NUM_EXPERTS, 1, D_OUT))
    out = _sc_combine_kernel()(ybuf, p0f, p1f, w0, w1)
    return out


# speedup vs baseline: 2.2606x; 2.2606x over previous
import jax
import jax.numpy as jnp
from jax.experimental import pallas as pl
from jax.experimental.pallas import tpu as pltpu

NUM_EXPERTS = 8
TOP_K = 2
N_TOKENS = 2048
D_IN = 768
D_OUT = 768


def _moe_body(x_ref, w_hbm, b_ref, gw_ref, gb_ref, gamma_ref, beta_ref,
              out_ref, wv_ref, c_ref, xbf_ref, sem):
    cp0 = pltpu.make_async_copy(w_hbm.at[0], wv_ref.at[0], sem.at[0])
    cp0.start()

    x = x_ref[...]
    logits = jnp.dot(x, gw_ref[...],
                     preferred_element_type=jnp.float32) + gb_ref[...]
    gx = jnp.sqrt(jnp.sum(logits * logits, axis=1, keepdims=True))
    nx = gx / (jnp.mean(gx, axis=0, keepdims=True) + 1e-06)
    logits = gamma_ref[...] * (logits * nx) + beta_ref[...] + logits
    m = jnp.max(logits, axis=1, keepdims=True)
    ex = jnp.exp(logits - m)
    gates = ex / jnp.sum(ex, axis=1, keepdims=True)
    ids = jax.lax.broadcasted_iota(jnp.int32, gates.shape, 1)
    m1 = jnp.max(gates, axis=1, keepdims=True)
    i1 = jnp.min(jnp.where(gates == m1, ids, NUM_EXPERTS),
                 axis=1, keepdims=True)
    sel1 = ids == i1
    g2 = jnp.where(sel1, -jnp.inf, gates)
    m2 = jnp.max(g2, axis=1, keepdims=True)
    i2 = jnp.min(jnp.where(g2 == m2, ids, NUM_EXPERTS),
                 axis=1, keepdims=True)
    sel2 = ids == i2
    c = jnp.where(sel1, m1, 0.0) + jnp.where(sel2, m2, 0.0)
    c_ref[...] = c
    xbf_ref[...] = x.astype(jnp.bfloat16)

    out_ref[...] = jnp.dot(c, b_ref[...], preferred_element_type=jnp.float32)
    for e in range(NUM_EXPERTS):
        if e + 1 < NUM_EXPERTS:
            pltpu.make_async_copy(w_hbm.at[e + 1], wv_ref.at[(e + 1) % 2],
                                  sem.at[(e + 1) % 2]).start()
        pltpu.make_async_copy(w_hbm.at[e], wv_ref.at[e % 2],
                              sem.at[e % 2]).wait()
        xs = c_ref[:, e:e + 1].astype(jnp.bfloat16) * xbf_ref[...]
        wbf = wv_ref[e % 2].astype(jnp.bfloat16)
        out_ref[...] += jnp.dot(xs, wbf, preferred_element_type=jnp.float32)


@jax.jit
def kernel(x, W, b, gate_W, gate_b, gamma, beta):
    return pl.pallas_call(
        _moe_body,
        grid=(1,),
        in_specs=[
            pl.BlockSpec((N_TOKENS, D_IN), lambda i: (0, 0)),
            pl.BlockSpec(memory_space=pl.ANY),
            pl.BlockSpec((NUM_EXPERTS, D_OUT), lambda i: (0, 0)),
            pl.BlockSpec((D_IN, NUM_EXPERTS), lambda i: (0, 0)),
            pl.BlockSpec((NUM_EXPERTS,), lambda i: (0,)),
            pl.BlockSpec((1, NUM_EXPERTS), lambda i: (0, 0)),
            pl.BlockSpec((1, NUM_EXPERTS), lambda i: (0, 0)),
        ],
        out_specs=pl.BlockSpec((N_TOKENS, D_OUT), lambda i: (0, 0)),
        out_shape=jax.ShapeDtypeStruct((N_TOKENS, D_OUT), jnp.float32),
        scratch_shapes=[
            pltpu.VMEM((2, D_IN, D_OUT), jnp.float32),
            pltpu.VMEM((N_TOKENS, NUM_EXPERTS), jnp.float32),
            pltpu.VMEM((N_TOKENS, D_IN), jnp.bfloat16),
            pltpu.SemaphoreType.DMA((2,)),
        ],
    )(x, W, b, gate_W, gate_b, gamma, beta)


# speedup vs baseline: 2.7138x; 1.2004x over previous
import jax
import jax.numpy as jnp
from jax.experimental import pallas as pl
from jax.experimental.pallas import tpu as pltpu

NUM_EXPERTS = 8
TOP_K = 2
N_TOKENS = 2048
D_IN = 768
D_OUT = 768


def _moe_body(x_ref, w_ref, b_ref, gw_ref, gb_ref, gamma_ref, beta_ref,
              out_ref, c_ref, xbf_ref):
    e = pl.program_id(0)

    @pl.when(e == 0)
    def _gating():
        x = x_ref[...]
        logits = jnp.dot(x, gw_ref[...],
                         preferred_element_type=jnp.float32) + gb_ref[...]
        gx = jnp.sqrt(jnp.sum(logits * logits, axis=1, keepdims=True))
        nx = gx / (jnp.mean(gx, axis=0, keepdims=True) + 1e-06)
        logits = gamma_ref[...] * (logits * nx) + beta_ref[...] + logits
        m = jnp.max(logits, axis=1, keepdims=True)
        ex = jnp.exp(logits - m)
        gates = ex / jnp.sum(ex, axis=1, keepdims=True)
        ids = jax.lax.broadcasted_iota(jnp.int32, gates.shape, 1)
        m1 = jnp.max(gates, axis=1, keepdims=True)
        i1 = jnp.min(jnp.where(gates == m1, ids, NUM_EXPERTS),
                     axis=1, keepdims=True)
        sel1 = ids == i1
        g2 = jnp.where(sel1, -jnp.inf, gates)
        m2 = jnp.max(g2, axis=1, keepdims=True)
        i2 = jnp.min(jnp.where(g2 == m2, ids, NUM_EXPERTS),
                     axis=1, keepdims=True)
        sel2 = ids == i2
        c = jnp.where(sel1, m1, 0.0) + jnp.where(sel2, m2, 0.0)
        c_ref[...] = c
        xbf_ref[...] = x.astype(jnp.bfloat16)
        out_ref[...] = jnp.dot(c, b_ref[...],
                               preferred_element_type=jnp.float32)

    cids = jax.lax.broadcasted_iota(jnp.int32, (N_TOKENS, NUM_EXPERTS), 1)
    ce = jnp.sum(jnp.where(cids == e, c_ref[...], 0.0), axis=1, keepdims=True)
    xs = ce.astype(jnp.bfloat16) * xbf_ref[...]
    wbf = w_ref[0].astype(jnp.bfloat16)
    out_ref[...] += jnp.dot(xs, wbf, preferred_element_type=jnp.float32)


@jax.jit
def kernel(x, W, b, gate_W, gate_b, gamma, beta):
    return pl.pallas_call(
        _moe_body,
        grid=(NUM_EXPERTS,),
        in_specs=[
            pl.BlockSpec((N_TOKENS, D_IN), lambda e: (0, 0)),
            pl.BlockSpec((1, D_IN, D_OUT), lambda e: (e, 0, 0)),
            pl.BlockSpec((NUM_EXPERTS, D_OUT), lambda e: (0, 0)),
            pl.BlockSpec((D_IN, NUM_EXPERTS), lambda e: (0, 0)),
            pl.BlockSpec((NUM_EXPERTS,), lambda e: (0,)),
            pl.BlockSpec((1, NUM_EXPERTS), lambda e: (0, 0)),
            pl.BlockSpec((1, NUM_EXPERTS), lambda e: (0, 0)),
        ],
        out_specs=pl.BlockSpec((N_TOKENS, D_OUT), lambda e: (0, 0)),
        out_shape=jax.ShapeDtypeStruct((N_TOKENS, D_OUT), jnp.float32),
        scratch_shapes=[
            pltpu.VMEM((N_TOKENS, NUM_EXPERTS), jnp.float32),
            pltpu.VMEM((N_TOKENS, D_IN), jnp.bfloat16),
        ],
    )(x, W, b, gate_W, gate_b, gamma, beta)
